# Initial kernel scaffold; baseline (speedup 1.0000x reference)
import jax
import jax.numpy as jnp
from jax.experimental import pallas as pl


def kernel(x, emb_table, fc1_w, fc1_b, fc2_w, fc2_b, bn1_gamma, bn1_beta, bn2_gamma, bn2_beta):
    raise NotImplementedError("write your pallas kernel here")


# speedup vs baseline: 2.4138x; 2.4138x over previous
import functools

import jax
import jax.numpy as jnp
from jax import lax
from jax.experimental import pallas as pl
from jax.experimental.pallas import tpu as pltpu
from jax.experimental.pallas import tpu_sc as plsc

VOCAB = 1000000
EMBED = 64
NUM_CLASS = 4
BATCH = 16384
HIST = 50
EPS = 1e-5

NC, NS, LANES = 2, 16, 16
NW = NC * NS
BPW = BATCH // NW
CHUNK = 16
NCHUNKS = BPW // CHUNK
KCOL = EMBED // LANES

_sc_mesh = plsc.VectorSubcoreMesh(core_axis_name="c", subcore_axis_name="s")


@functools.partial(
    pl.kernel,
    out_type=jax.ShapeDtypeStruct((BATCH, EMBED), jnp.float32),
    mesh=_sc_mesh,
    scratch_types=[
        pltpu.VMEM((CHUNK, HIST), jnp.int32),
        pltpu.VMEM((CHUNK, HIST, EMBED), jnp.float32),
        pltpu.VMEM((CHUNK, EMBED), jnp.float32),
        pltpu.SemaphoreType.DMA,
    ],
    compiler_params=pltpu.CompilerParams(use_tc_tiling_on_sc=False),
)
def _bag_sums(x_hbm, table_hbm, out_hbm, idx_v, rows_v, out_v, sem):
    wid = lax.axis_index("s") * NC + lax.axis_index("c")

    def chunk_body(ci, carry):
        base = wid * BPW + ci * CHUNK
        pltpu.sync_copy(x_hbm.at[pl.ds(base, CHUNK)], idx_v)
        copies = [
            pltpu.async_copy(table_hbm.at[idx_v.at[b]], rows_v.at[b], sem)
            for b in range(CHUNK)
        ]
        for c in copies:
            c.wait()
        for b in range(CHUNK):
            def rbody(r, accs, b=b):
                return tuple(
                    accs[k] + rows_v[b, r, pl.ds(k * LANES, LANES)]
                    for k in range(KCOL)
                )
            accs = lax.fori_loop(
                0, HIST, rbody,
                tuple(jnp.zeros((LANES,), jnp.float32) for _ in range(KCOL)),
            )
            for k in range(KCOL):
                out_v[b, pl.ds(k * LANES, LANES)] = accs[k]
        pltpu.sync_copy(out_v, out_hbm.at[pl.ds(base, CHUNK)])
        return carry

    lax.fori_loop(0, NCHUNKS, chunk_body, 0)


def _mlp_body(bag_ref, s1_ref, b1_ref, w1_ref, s2_ref, b2_ref, w2_ref,
              fc2b_ref, out_ref):
    h = bag_ref[...] * s1_ref[...] + b1_ref[...]
    h = jnp.maximum(h, 0.0)
    h = jnp.dot(h, w1_ref[...], preferred_element_type=jnp.float32)
    h = h * s2_ref[...] + b2_ref[...]
    h = jnp.maximum(h, 0.0)
    out_ref[...] = (
        jnp.dot(h, w2_ref[...], preferred_element_type=jnp.float32)
        + fc2b_ref[...]
    )


_BM = 2048


def _mlp(sums, s1, b1, w1, s2, b2, w2, fc2b):
    grid = (BATCH // _BM,)
    return pl.pallas_call(
        _mlp_body,
        grid=grid,
        in_specs=[
            pl.BlockSpec((_BM, EMBED), lambda i: (i, 0)),
            pl.BlockSpec((1, EMBED), lambda i: (0, 0)),
            pl.BlockSpec((1, EMBED), lambda i: (0, 0)),
            pl.BlockSpec((EMBED, 128), lambda i: (0, 0)),
            pl.BlockSpec((1, 128), lambda i: (0, 0)),
            pl.BlockSpec((1, 128), lambda i: (0, 0)),
            pl.BlockSpec((128, NUM_CLASS), lambda i: (0, 0)),
            pl.BlockSpec((1, NUM_CLASS), lambda i: (0, 0)),
        ],
        out_specs=pl.BlockSpec((_BM, NUM_CLASS), lambda i: (i, 0)),
        out_shape=jax.ShapeDtypeStruct((BATCH, NUM_CLASS), jnp.float32),
    )(sums, s1, b1, w1, s2, b2, w2, fc2b)


def kernel(x, emb_table, fc1_w, fc1_b, fc2_w, fc2_b,
           bn1_gamma, bn1_beta, bn2_gamma, bn2_beta):
    sums = _bag_sums(x.astype(jnp.int32), emb_table)
    inv = 1.0 / jnp.sqrt(1.0 + EPS)
    s1 = (bn1_gamma * inv / HIST).reshape(1, EMBED)
    b1 = bn1_beta.reshape(1, EMBED)
    s2 = (bn2_gamma * inv).reshape(1, 128)
    b2 = (fc1_b * bn2_gamma * inv + bn2_beta).reshape(1, 128)
    return _mlp(sums, s1, b1, fc1_w.T, s2, b2, fc2_w.T,
                fc2_b.reshape(1, NUM_CLASS))


# speedup vs baseline: 2.7473x; 1.1382x over previous
import functools

import jax
import jax.numpy as jnp
from jax import lax
from jax.experimental import pallas as pl
from jax.experimental.pallas import tpu as pltpu
from jax.experimental.pallas import tpu_sc as plsc

VOCAB = 1000000
EMBED = 64
NUM_CLASS = 4
BATCH = 16384
HIST = 50
EPS = 1e-5

NC, NS, LANES = 2, 16, 16
NW = NC * NS
BPW = BATCH // NW
CHUNK = 16
NCHUNKS = BPW // CHUNK
KCOL = EMBED // LANES
CIDX = CHUNK * HIST

_sc_mesh = plsc.VectorSubcoreMesh(core_axis_name="c", subcore_axis_name="s")


@functools.partial(
    pl.kernel,
    out_type=jax.ShapeDtypeStruct((BATCH, EMBED), jnp.float32),
    mesh=_sc_mesh,
    scratch_types=[
        pltpu.VMEM((CIDX,), jnp.int32),
        pltpu.VMEM((CIDX,), jnp.int32),
        pltpu.VMEM((CIDX, EMBED), jnp.float32),
        pltpu.VMEM((CIDX, EMBED), jnp.float32),
        pltpu.VMEM((CHUNK, EMBED), jnp.float32),
        pltpu.VMEM((CHUNK, EMBED), jnp.float32),
        pltpu.SemaphoreType.DMA,
    ],
    compiler_params=pltpu.CompilerParams(use_tc_tiling_on_sc=False),
)
def _bag_sums(xf_hbm, table_hbm, out_hbm,
              idx0, idx1, rows0, rows1, out0, out1, gsem):
    wid = lax.axis_index("s") * NC + lax.axis_index("c")
    idx = (idx0, idx1)
    rows = (rows0, rows1)
    outs = (out0, out1)

    def fire(ci, slot):
        base = wid * BPW + ci * CHUNK
        pltpu.sync_copy(xf_hbm.at[pl.ds(base * HIST, CIDX)], idx[slot])
        pltpu.async_copy(table_hbm.at[idx[slot]], rows[slot], gsem)

    def drain_reduce_store(ci, slot):
        base = wid * BPW + ci * CHUNK
        pltpu.make_async_copy(table_hbm.at[idx[slot]], rows[slot], gsem).wait()
        rv = rows[slot]
        ov = outs[slot]

        def bag_body(b, carry):
            accs = [rv[b * HIST, pl.ds(k * LANES, LANES)] for k in range(KCOL)]
            for r in range(1, HIST):
                for k in range(KCOL):
                    accs[k] = accs[k] + rv[b * HIST + r, pl.ds(k * LANES, LANES)]
            for k in range(KCOL):
                ov[b, pl.ds(k * LANES, LANES)] = accs[k]
            return carry

        lax.fori_loop(0, CHUNK, bag_body, 0)
        pltpu.sync_copy(ov, out_hbm.at[pl.ds(base, CHUNK)])

    fire(0, 0)

    def pair_body(i, carry):
        c0 = 2 * i
        fire(c0 + 1, 1)
        drain_reduce_store(c0, 0)

        @pl.when(c0 + 2 < NCHUNKS)
        def _():
            fire(c0 + 2, 0)

        drain_reduce_store(c0 + 1, 1)
        return carry

    lax.fori_loop(0, NCHUNKS // 2, pair_body, 0)


def _mlp_body(bag_ref, s1_ref, b1_ref, w1_ref, s2_ref, b2_ref, w2_ref,
              fc2b_ref, out_ref):
    h = bag_ref[...] * s1_ref[...] + b1_ref[...]
    h = jnp.maximum(h, 0.0)
    h = jnp.dot(h, w1_ref[...], preferred_element_type=jnp.float32)
    h = h * s2_ref[...] + b2_ref[...]
    h = jnp.maximum(h, 0.0)
    out_ref[...] = (
        jnp.dot(h, w2_ref[...], preferred_element_type=jnp.float32)
        + fc2b_ref[...]
    )


_BM = 2048


def _mlp(sums, s1, b1, w1, s2, b2, w2, fc2b):
    grid = (BATCH // _BM,)
    return pl.pallas_call(
        _mlp_body,
        grid=grid,
        in_specs=[
            pl.BlockSpec((_BM, EMBED), lambda i: (i, 0)),
            pl.BlockSpec((1, EMBED), lambda i: (0, 0)),
            pl.BlockSpec((1, EMBED), lambda i: (0, 0)),
            pl.BlockSpec((EMBED, 128), lambda i: (0, 0)),
            pl.BlockSpec((1, 128), lambda i: (0, 0)),
            pl.BlockSpec((1, 128), lambda i: (0, 0)),
            pl.BlockSpec((128, NUM_CLASS), lambda i: (0, 0)),
            pl.BlockSpec((1, NUM_CLASS), lambda i: (0, 0)),
        ],
        out_specs=pl.BlockSpec((_BM, NUM_CLASS), lambda i: (i, 0)),
        out_shape=jax.ShapeDtypeStruct((BATCH, NUM_CLASS), jnp.float32),
    )(sums, s1, b1, w1, s2, b2, w2, fc2b)


def kernel(x, emb_table, fc1_w, fc1_b, fc2_w, fc2_b,
           bn1_gamma, bn1_beta, bn2_gamma, bn2_beta):
    xf = x.astype(jnp.int32).reshape(BATCH * HIST)
    sums = _bag_sums(xf, emb_table)
    inv = 1.0 / jnp.sqrt(1.0 + EPS)
    s1 = (bn1_gamma * inv / HIST).reshape(1, EMBED)
    b1 = bn1_beta.reshape(1, EMBED)
    s2 = (bn2_gamma * inv).reshape(1, 128)
    b2 = (fc1_b * bn2_gamma * inv + bn2_beta).reshape(1, 128)
    return _mlp(sums, s1, b1, fc1_w.T, s2, b2, fc2_w.T,
                fc2_b.reshape(1, NUM_CLASS))
